# baseline (speedup 1.0000x reference)
import functools

import jax
import jax.numpy as jnp
from jax import lax
from jax.experimental import pallas as pl
from jax.experimental.pallas import tpu as pltpu
from jax.experimental.pallas import tpu_sc as plsc

N = 10000
NP = 10240
E = 320000
EP = 327680
D = 128
STEPS = 6

CHUNK = 128
NCHUNKS = EP // CHUNK
NWORK = 32
CPW = NCHUNKS // NWORK
RPS = NP // 16

BN = 1024
BE = 1280

_f32 = jnp.float32
_bf16 = jnp.bfloat16


def _dot(a, b):
    return lax.dot_general(a, b, (((1,), (0,)), ((), ())),
                           preferred_element_type=_f32)


def _ln(z):
    m = jnp.mean(z, axis=-1, keepdims=True)
    zc = z - m
    v = jnp.mean(zc * zc, axis=-1, keepdims=True)
    return zc * lax.rsqrt(v + 1e-5)


def _sc_mesh():
    return plsc.VectorSubcoreMesh(core_axis_name="c", subcore_axis_name="s",
                                  num_cores=2)


def _sc_gather(hs, hr, send2, recv2):
    @functools.partial(
        pl.kernel, mesh=_sc_mesh(),
        out_type=(jax.ShapeDtypeStruct((EP, D), _f32),
                  jax.ShapeDtypeStruct((EP, D), _f32)),
        scratch_types=[pltpu.VMEM((CPW, CHUNK), jnp.int32),
                       pltpu.VMEM((CPW, CHUNK), jnp.int32),
                       pltpu.VMEM((CHUNK, D), _f32),
                       pltpu.VMEM((CHUNK, D), _f32),
                       pltpu.VMEM((CHUNK, D), _f32),
                       pltpu.VMEM((CHUNK, D), _f32),
                       pltpu.SemaphoreType.DMA,
                       pltpu.SemaphoreType.DMA,
                       pltpu.SemaphoreType.DMA,
                       pltpu.SemaphoreType.DMA])
    def k(hs_hbm, hr_hbm, send_hbm, recv_hbm, outs_hbm, outr_hbm,
          idxs, idxr, bs0, br0, bs1, br1, sg0, sg1, sw0, sw1):
        wid = lax.axis_index("s") * 2 + lax.axis_index("c")
        c0 = wid * CPW
        pltpu.sync_copy(send_hbm.at[pl.ds(c0, CPW)], idxs)
        pltpu.sync_copy(recv_hbm.at[pl.ds(c0, CPW)], idxr)
        slots = ((bs0, br0, sg0, sw0), (bs1, br1, sg1, sw1))

        def start_gather(j, s):
            bs, br, sg, _ = slots[s]
            pltpu.async_copy(hs_hbm.at[idxs.at[j]], bs, sg)
            pltpu.async_copy(hr_hbm.at[idxr.at[j]], br, sg)

        def wait_gather(s):
            bs, br, sg, _ = slots[s]
            pltpu.make_async_copy(hs_hbm.at[pl.ds(0, CHUNK)], bs, sg).wait()
            pltpu.make_async_copy(hr_hbm.at[pl.ds(0, CHUNK)], br, sg).wait()

        def start_wb(j, s):
            bs, br, _, sw = slots[s]
            base = (c0 + j) * CHUNK
            pltpu.async_copy(bs, outs_hbm.at[pl.ds(base, CHUNK)], sw)
            pltpu.async_copy(br, outr_hbm.at[pl.ds(base, CHUNK)], sw)

        def wait_wb(s):
            bs, br, _, sw = slots[s]
            pltpu.make_async_copy(hs_hbm.at[pl.ds(0, CHUNK)], bs, sw).wait()
            pltpu.make_async_copy(hs_hbm.at[pl.ds(0, CHUNK)], br, sw).wait()

        start_gather(0, 0)

        @pl.loop(0, CPW // 2)
        def _(p):
            j = p * 2

            @pl.when(p > 0)
            def _():
                wait_wb(1)

            start_gather(j + 1, 1)
            wait_gather(0)
            start_wb(j, 0)

            @pl.when(p < CPW // 2 - 1)
            def _():
                wait_wb(0)
                start_gather(j + 2, 0)

            wait_gather(1)
            start_wb(j + 1, 1)

        wait_wb(0)
        wait_wb(1)

    return k(hs, hr, send2, recv2)


def _sc_scatter(he, recv2, zeros):
    @functools.partial(
        pl.kernel, mesh=_sc_mesh(),
        out_type=jax.ShapeDtypeStruct((2, NP, D), _f32),
        scratch_types=[pltpu.VMEM((CPW, CHUNK), jnp.int32),
                       pltpu.VMEM((CHUNK, D), _f32),
                       pltpu.VMEM((CHUNK, D), _f32),
                       pltpu.VMEM_SHARED((NP, D), _f32),
                       pltpu.SemaphoreType.DMA,
                       pltpu.SemaphoreType.DMA,
                       pltpu.SemaphoreType.DMA,
                       pltpu.SemaphoreType.DMA])
    def k(he_hbm, recv_hbm, z_hbm, out_hbm, idx, rb0, rb1, acc,
          sl0, sl1, ss0, ss1):
        core = lax.axis_index("c")
        sid = lax.axis_index("s")
        wid = sid * 2 + core
        c0 = wid * CPW
        pltpu.sync_copy(recv_hbm.at[pl.ds(c0, CPW)], idx)
        pltpu.sync_copy(z_hbm.at[pl.ds(sid * RPS, RPS)],
                        acc.at[pl.ds(sid * RPS, RPS)])
        plsc.subcore_barrier()
        slots = ((rb0, sl0, ss0), (rb1, sl1, ss1))

        def start_load(j, s):
            rb, sl, _ = slots[s]
            pltpu.async_copy(he_hbm.at[pl.ds((c0 + j) * CHUNK, CHUNK)], rb, sl)

        def wait_load(s):
            rb, sl, _ = slots[s]
            pltpu.make_async_copy(he_hbm.at[pl.ds(0, CHUNK)], rb, sl).wait()

        def start_scat(j, s):
            rb, _, ss = slots[s]
            pltpu.async_copy(rb, acc.at[idx.at[j]], ss, add=True)

        def wait_scat(s):
            rb, _, ss = slots[s]
            pltpu.make_async_copy(he_hbm.at[pl.ds(0, CHUNK)], rb, ss).wait()

        start_load(0, 0)

        @pl.loop(0, CPW // 2)
        def _(p):
            j = p * 2

            @pl.when(p > 0)
            def _():
                wait_scat(1)

            start_load(j + 1, 1)
            wait_load(0)
            start_scat(j, 0)

            @pl.when(p < CPW // 2 - 1)
            def _():
                wait_scat(0)
                start_load(j + 2, 0)

            wait_load(1)
            start_scat(j + 1, 1)

        wait_scat(0)
        wait_scat(1)
        plsc.subcore_barrier()
        pltpu.sync_copy(acc.at[pl.ds(sid * RPS, RPS)],
                        out_hbm.at[core, pl.ds(sid * RPS, RPS)])

    return k(he, recv2, zeros)


_WSPEC = pl.BlockSpec((D, D), lambda i: (0, 0))
_BSPEC = pl.BlockSpec((1, D), lambda i: (0, 0))
_CP = pltpu.CompilerParams(dimension_semantics=("parallel",))


def _row_spec(rows):
    return pl.BlockSpec((rows, D), lambda i: (i, 0))


def _enc_node(x, A0, a0, A1, a1, A2, a2, W1s, W1r):
    def body(x_r, A0_r, a0_r, A1_r, a1_r, A2_r, a2_r, Ws_r, Wr_r,
             hn_r, hs_r, hr_r):
        h = jnp.maximum(_dot(x_r[...], A0_r[...]) + a0_r[...], 0.0)
        h = jnp.maximum(_dot(h, A1_r[...]) + a1_r[...], 0.0)
        h = _dot(h, A2_r[...]) + a2_r[...]
        hn = _ln(h)
        hn_r[...] = hn
        hs_r[...] = _dot(hn, Ws_r[...])
        hr_r[...] = _dot(hn, Wr_r[...])

    return pl.pallas_call(
        body,
        grid=(NP // BN,),
        in_specs=[_row_spec(BN), _WSPEC, _BSPEC, _WSPEC, _BSPEC, _WSPEC,
                  _BSPEC, _WSPEC, _WSPEC],
        out_specs=[_row_spec(BN)] * 3,
        out_shape=[jax.ShapeDtypeStruct((NP, D), _f32)] * 3,
        compiler_params=_CP,
    )(x, A0, a0, A1, a1, A2, a2, W1s, W1r)


def _enc_edge(ea, E0, e0, E1, e1, E2, e2):
    def body(ea_r, E0_r, e0_r, E1_r, e1_r, E2_r, e2_r, he_r):
        h = jnp.maximum(_dot(ea_r[...], E0_r[...]) + e0_r[...], 0.0)
        h = jnp.maximum(_dot(h, E1_r[...]) + e1_r[...], 0.0)
        h = _dot(h, E2_r[...]) + e2_r[...]
        he_r[...] = _ln(h)

    return pl.pallas_call(
        body,
        grid=(EP // BE,),
        in_specs=[pl.BlockSpec((BE, 8), lambda i: (i, 0)),
                  pl.BlockSpec((8, D), lambda i: (0, 0)), _BSPEC,
                  _WSPEC, _BSPEC, _WSPEC, _BSPEC],
        out_specs=[_row_spec(BE)],
        out_shape=[jax.ShapeDtypeStruct((EP, D), _f32)],
        compiler_params=_CP,
    )(ea, E0, e0, E1, e1, E2, e2)[0]


def _edge_step(he, tS, tR, W1e, b1, W2, b2, W3, b3):
    def body(he_r, tS_r, tR_r, W1_r, b1_r, W2_r, b2_r, W3_r, b3_r, out_r):
        he_v = he_r[...]
        z = jnp.maximum(_dot(he_v, W1_r[...]) + tS_r[...] + tR_r[...]
                        + b1_r[...], 0.0)
        z = jnp.maximum(_dot(z, W2_r[...]) + b2_r[...], 0.0)
        z = _dot(z, W3_r[...]) + b3_r[...]
        out_r[...] = he_v + _ln(z)

    return pl.pallas_call(
        body,
        grid=(EP // BE,),
        in_specs=[_row_spec(BE), _row_spec(BE), _row_spec(BE),
                  _WSPEC, _BSPEC, _WSPEC, _BSPEC, _WSPEC, _BSPEC],
        out_specs=[_row_spec(BE)],
        out_shape=[jax.ShapeDtypeStruct((EP, D), _f32)],
        compiler_params=_CP,
    )(he, tS, tR, W1e, b1, W2, b2, W3, b3)[0]


def _node_step(hn, p0, p1, V1n, V1a, c1, V2, c2, V3, c3, W1s, W1r):
    def body(hn_r, p0_r, p1_r, V1n_r, V1a_r, c1_r, V2_r, c2_r, V3_r, c3_r,
             Ws_r, Wr_r, out_r, hs_r, hr_r):
        hn_v = hn_r[...]
        agg = p0_r[...] + p1_r[...]
        z = jnp.maximum(_dot(hn_v, V1n_r[...]) + _dot(agg, V1a_r[...])
                        + c1_r[...], 0.0)
        z = jnp.maximum(_dot(z, V2_r[...]) + c2_r[...], 0.0)
        z = _dot(z, V3_r[...]) + c3_r[...]
        hn_new = hn_v + _ln(z)
        out_r[...] = hn_new
        hs_r[...] = _dot(hn_new, Ws_r[...])
        hr_r[...] = _dot(hn_new, Wr_r[...])

    return pl.pallas_call(
        body,
        grid=(NP // BN,),
        in_specs=[_row_spec(BN), _row_spec(BN), _row_spec(BN),
                  _WSPEC, _WSPEC, _BSPEC, _WSPEC, _BSPEC, _WSPEC, _BSPEC,
                  _WSPEC, _WSPEC],
        out_specs=[_row_spec(BN)] * 3,
        out_shape=[jax.ShapeDtypeStruct((NP, D), _f32)] * 3,
        compiler_params=_CP,
    )(hn, p0, p1, V1n, V1a, c1, V2, c2, V3, c3, W1s, W1r)


def _dec(hn, D0, d0, D1, d1, D2, d2):
    def body(hn_r, D0_r, d0_r, D1_r, d1_r, D2_r, d2_r, out_r):
        h = jnp.maximum(_dot(hn_r[...], D0_r[...]) + d0_r[...], 0.0)
        h = jnp.maximum(_dot(h, D1_r[...]) + d1_r[...], 0.0)
        out_r[...] = _dot(h, D2_r[...]) + d2_r[...]

    return pl.pallas_call(
        body,
        grid=(NP // BN,),
        in_specs=[_row_spec(BN), _WSPEC, _BSPEC, _WSPEC, _BSPEC, _WSPEC,
                  _BSPEC],
        out_specs=[_row_spec(BN)],
        out_shape=[jax.ShapeDtypeStruct((NP, D), _f32)],
        compiler_params=_CP,
    )(hn, D0, d0, D1, d1, D2, d2)[0]


def _b(v):
    return v.reshape(1, D)


def kernel(x, edge_index, edge_attr, params):
    pad_e = EP - E
    epad = N + (jnp.arange(pad_e, dtype=jnp.int32) % (NP - N))
    send2 = jnp.concatenate([edge_index[0], epad]).reshape(NCHUNKS, CHUNK)
    recv2 = jnp.concatenate([edge_index[1], epad]).reshape(NCHUNKS, CHUNK)
    xp = jnp.pad(x, ((0, NP - N), (0, 0)))
    eap = jnp.pad(edge_attr, ((0, pad_e), (0, 4)))
    zeros = jnp.zeros((NP, D), _f32)

    (A0, a0), (A1, a1), (A2, a2) = params["enc_node"]
    (E0, e0), (E1, e1), (E2, e2) = params["enc_edge"]
    (D0, d0), (D1, d1), (D2, d2) = params["dec"]
    E0p = jnp.pad(E0, ((0, 4), (0, 0)))

    eW = []
    for s in range(STEPS):
        (W1, b1), (W2, b2), (W3, b3) = params["edge_mlp"][s]
        eW.append((W1[:D], W1[D:2 * D], W1[2 * D:], _b(b1), W2, _b(b2), W3,
                   _b(b3)))
    nW = []
    for s in range(STEPS):
        (V1, c1), (V2, c2), (V3, c3) = params["node_mlp"][s]
        nW.append((V1[:D], V1[D:], _b(c1), V2, _b(c2), V3, _b(c3)))

    hn, hs, hr = _enc_node(xp, A0, _b(a0), A1, _b(a1), A2, _b(a2),
                           eW[0][1], eW[0][2])
    he = _enc_edge(eap, E0p, _b(e0), E1, _b(e1), E2, _b(e2))

    for s in range(STEPS):
        W1e, _, _, b1, W2, b2, W3, b3 = eW[s]
        tS, tR = _sc_gather(hs, hr, send2, recv2)
        he = _edge_step(he, tS, tR, W1e, b1, W2, b2, W3, b3)
        partials = _sc_scatter(he, recv2, zeros)
        V1n, V1a, c1, V2, c2, V3, c3 = nW[s]
        nxt = eW[s + 1] if s + 1 < STEPS else eW[0]
        hn, hs, hr = _node_step(hn, partials[0], partials[1],
                                V1n, V1a, c1, V2, c2, V3, c3,
                                nxt[1], nxt[2])

    out = _dec(hn, D0, _b(d0), D1, _b(d1), D2, _b(d2))
    return out[:N]


# speedup vs baseline: 1.0509x; 1.0509x over previous
import functools

import jax
import jax.numpy as jnp
from jax import lax
from jax.experimental import pallas as pl
from jax.experimental.pallas import tpu as pltpu
from jax.experimental.pallas import tpu_sc as plsc

N = 10000
NP = 10240
E = 320000
EP = 327680
D = 128
STEPS = 6

EH = EP // 2
CHUNK = 128
NCHUNKS = EH // CHUNK
NWORK = 32
CPW = NCHUNKS // NWORK
RPS = NP // 16

BN = 1024
BE = 1280

_f32 = jnp.float32
_bf16 = jnp.bfloat16


def _dot(a, b):
    return lax.dot_general(a, b, (((1,), (0,)), ((), ())),
                           preferred_element_type=_f32)


def _ln(z):
    m = jnp.mean(z, axis=-1, keepdims=True)
    zc = z - m
    v = jnp.mean(zc * zc, axis=-1, keepdims=True)
    return zc * lax.rsqrt(v + 1e-5)


def _sc_mesh():
    return plsc.VectorSubcoreMesh(core_axis_name="c", subcore_axis_name="s",
                                  num_cores=2)


def _sc_gather(hs, hr, send2, recv2):
    @functools.partial(
        pl.kernel, mesh=_sc_mesh(),
        out_type=(jax.ShapeDtypeStruct((EH, D), _f32),
                  jax.ShapeDtypeStruct((EH, D), _f32)),
        scratch_types=[pltpu.VMEM((CPW, CHUNK), jnp.int32),
                       pltpu.VMEM((CPW, CHUNK), jnp.int32),
                       pltpu.VMEM((CHUNK, D), _f32),
                       pltpu.VMEM((CHUNK, D), _f32),
                       pltpu.VMEM((CHUNK, D), _f32),
                       pltpu.VMEM((CHUNK, D), _f32),
                       pltpu.SemaphoreType.DMA,
                       pltpu.SemaphoreType.DMA,
                       pltpu.SemaphoreType.DMA,
                       pltpu.SemaphoreType.DMA])
    def k(hs_hbm, hr_hbm, send_hbm, recv_hbm, outs_hbm, outr_hbm,
          idxs, idxr, bs0, br0, bs1, br1, sg0, sg1, sw0, sw1):
        wid = lax.axis_index("s") * 2 + lax.axis_index("c")
        c0 = wid * CPW
        pltpu.sync_copy(send_hbm.at[pl.ds(c0, CPW)], idxs)
        pltpu.sync_copy(recv_hbm.at[pl.ds(c0, CPW)], idxr)
        slots = ((bs0, br0, sg0, sw0), (bs1, br1, sg1, sw1))

        def start_gather(j, s):
            bs, br, sg, _ = slots[s]
            pltpu.async_copy(hs_hbm.at[idxs.at[j]], bs, sg)
            pltpu.async_copy(hr_hbm.at[idxr.at[j]], br, sg)

        def wait_gather(s):
            bs, br, sg, _ = slots[s]
            pltpu.make_async_copy(hs_hbm.at[pl.ds(0, CHUNK)], bs, sg).wait()
            pltpu.make_async_copy(hr_hbm.at[pl.ds(0, CHUNK)], br, sg).wait()

        def start_wb(j, s):
            bs, br, _, sw = slots[s]
            base = (c0 + j) * CHUNK
            pltpu.async_copy(bs, outs_hbm.at[pl.ds(base, CHUNK)], sw)
            pltpu.async_copy(br, outr_hbm.at[pl.ds(base, CHUNK)], sw)

        def wait_wb(s):
            bs, br, _, sw = slots[s]
            pltpu.make_async_copy(hs_hbm.at[pl.ds(0, CHUNK)], bs, sw).wait()
            pltpu.make_async_copy(hs_hbm.at[pl.ds(0, CHUNK)], br, sw).wait()

        start_gather(0, 0)

        @pl.loop(0, CPW // 2)
        def _(p):
            j = p * 2

            @pl.when(p > 0)
            def _():
                wait_wb(1)

            start_gather(j + 1, 1)
            wait_gather(0)
            start_wb(j, 0)

            @pl.when(p < CPW // 2 - 1)
            def _():
                wait_wb(0)
                start_gather(j + 2, 0)

            wait_gather(1)
            start_wb(j + 1, 1)

        wait_wb(0)
        wait_wb(1)

    return k(hs, hr, send2, recv2)


def _sc_scatter(he, recv2, zeros):
    @functools.partial(
        pl.kernel, mesh=_sc_mesh(),
        out_type=jax.ShapeDtypeStruct((2, NP, D), _f32),
        scratch_types=[pltpu.VMEM((CPW, CHUNK), jnp.int32),
                       pltpu.VMEM((CHUNK, D), _f32),
                       pltpu.VMEM((CHUNK, D), _f32),
                       pltpu.VMEM_SHARED((NP, D), _f32),
                       pltpu.SemaphoreType.DMA,
                       pltpu.SemaphoreType.DMA,
                       pltpu.SemaphoreType.DMA,
                       pltpu.SemaphoreType.DMA])
    def k(he_hbm, recv_hbm, z_hbm, out_hbm, idx, rb0, rb1, acc,
          sl0, sl1, ss0, ss1):
        core = lax.axis_index("c")
        sid = lax.axis_index("s")
        wid = sid * 2 + core
        c0 = wid * CPW
        pltpu.sync_copy(recv_hbm.at[pl.ds(c0, CPW)], idx)
        pltpu.sync_copy(z_hbm.at[pl.ds(sid * RPS, RPS)],
                        acc.at[pl.ds(sid * RPS, RPS)])
        plsc.subcore_barrier()
        slots = ((rb0, sl0, ss0), (rb1, sl1, ss1))

        def start_load(j, s):
            rb, sl, _ = slots[s]
            pltpu.async_copy(he_hbm.at[pl.ds((c0 + j) * CHUNK, CHUNK)], rb, sl)

        def wait_load(s):
            rb, sl, _ = slots[s]
            pltpu.make_async_copy(he_hbm.at[pl.ds(0, CHUNK)], rb, sl).wait()

        def start_scat(j, s):
            rb, _, ss = slots[s]
            pltpu.async_copy(rb, acc.at[idx.at[j]], ss, add=True)

        def wait_scat(s):
            rb, _, ss = slots[s]
            pltpu.make_async_copy(he_hbm.at[pl.ds(0, CHUNK)], rb, ss).wait()

        start_load(0, 0)

        @pl.loop(0, CPW // 2)
        def _(p):
            j = p * 2

            @pl.when(p > 0)
            def _():
                wait_scat(1)

            start_load(j + 1, 1)
            wait_load(0)
            start_scat(j, 0)

            @pl.when(p < CPW // 2 - 1)
            def _():
                wait_scat(0)
                start_load(j + 2, 0)

            wait_load(1)
            start_scat(j + 1, 1)

        wait_scat(0)
        wait_scat(1)
        plsc.subcore_barrier()
        pltpu.sync_copy(acc.at[pl.ds(sid * RPS, RPS)],
                        out_hbm.at[core, pl.ds(sid * RPS, RPS)])

    return k(he, recv2, zeros)


_WSPEC = pl.BlockSpec((D, D), lambda i: (0, 0))
_BSPEC = pl.BlockSpec((1, D), lambda i: (0, 0))
_CP = pltpu.CompilerParams(dimension_semantics=("parallel",))


def _row_spec(rows):
    return pl.BlockSpec((rows, D), lambda i: (i, 0))


def _enc_node(x, A0, a0, A1, a1, A2, a2, W1s, W1r):
    def body(x_r, A0_r, a0_r, A1_r, a1_r, A2_r, a2_r, Ws_r, Wr_r,
             hn_r, hs_r, hr_r):
        h = jnp.maximum(_dot(x_r[...], A0_r[...]) + a0_r[...], 0.0)
        h = jnp.maximum(_dot(h, A1_r[...]) + a1_r[...], 0.0)
        h = _dot(h, A2_r[...]) + a2_r[...]
        hn = _ln(h)
        hn_r[...] = hn
        hs_r[...] = _dot(hn, Ws_r[...])
        hr_r[...] = _dot(hn, Wr_r[...])

    return pl.pallas_call(
        body,
        grid=(NP // BN,),
        in_specs=[_row_spec(BN), _WSPEC, _BSPEC, _WSPEC, _BSPEC, _WSPEC,
                  _BSPEC, _WSPEC, _WSPEC],
        out_specs=[_row_spec(BN)] * 3,
        out_shape=[jax.ShapeDtypeStruct((NP, D), _f32)] * 3,
        compiler_params=_CP,
    )(x, A0, a0, A1, a1, A2, a2, W1s, W1r)


def _enc_edge(ea, E0, e0, E1, e1, E2, e2):
    def body(ea_r, E0_r, e0_r, E1_r, e1_r, E2_r, e2_r, he_r):
        h = jnp.maximum(_dot(ea_r[...], E0_r[...]) + e0_r[...], 0.0)
        h = jnp.maximum(_dot(h, E1_r[...]) + e1_r[...], 0.0)
        h = _dot(h, E2_r[...]) + e2_r[...]
        he_r[...] = _ln(h)

    return pl.pallas_call(
        body,
        grid=(EH // BE,),
        in_specs=[pl.BlockSpec((BE, 8), lambda i: (i, 0)),
                  pl.BlockSpec((8, D), lambda i: (0, 0)), _BSPEC,
                  _WSPEC, _BSPEC, _WSPEC, _BSPEC],
        out_specs=[_row_spec(BE)],
        out_shape=[jax.ShapeDtypeStruct((EH, D), _f32)],
        compiler_params=_CP,
    )(ea, E0, e0, E1, e1, E2, e2)[0]


def _edge_step(he, tS, tR, W1e, b1, W2, b2, W3, b3):
    def body(he_r, tS_r, tR_r, W1_r, b1_r, W2_r, b2_r, W3_r, b3_r, out_r):
        he_v = he_r[...]
        z = jnp.maximum(_dot(he_v, W1_r[...]) + tS_r[...] + tR_r[...]
                        + b1_r[...], 0.0)
        z = jnp.maximum(_dot(z, W2_r[...]) + b2_r[...], 0.0)
        z = _dot(z, W3_r[...]) + b3_r[...]
        out_r[...] = he_v + _ln(z)

    return pl.pallas_call(
        body,
        grid=(EH // BE,),
        in_specs=[_row_spec(BE), _row_spec(BE), _row_spec(BE),
                  _WSPEC, _BSPEC, _WSPEC, _BSPEC, _WSPEC, _BSPEC],
        out_specs=[_row_spec(BE)],
        out_shape=[jax.ShapeDtypeStruct((EH, D), _f32)],
        compiler_params=_CP,
    )(he, tS, tR, W1e, b1, W2, b2, W3, b3)[0]


def _node_step(hn, pA, pB, V1n, V1a, c1, V2, c2, V3, c3, W1s, W1r):
    def body(hn_r, pa0_r, pa1_r, pb0_r, pb1_r, V1n_r, V1a_r, c1_r, V2_r,
             c2_r, V3_r, c3_r, Ws_r, Wr_r, out_r, hs_r, hr_r):
        hn_v = hn_r[...]
        agg = (pa0_r[...] + pa1_r[...]) + (pb0_r[...] + pb1_r[...])
        z = jnp.maximum(_dot(hn_v, V1n_r[...]) + _dot(agg, V1a_r[...])
                        + c1_r[...], 0.0)
        z = jnp.maximum(_dot(z, V2_r[...]) + c2_r[...], 0.0)
        z = _dot(z, V3_r[...]) + c3_r[...]
        hn_new = hn_v + _ln(z)
        out_r[...] = hn_new
        hs_r[...] = _dot(hn_new, Ws_r[...])
        hr_r[...] = _dot(hn_new, Wr_r[...])

    return pl.pallas_call(
        body,
        grid=(NP // BN,),
        in_specs=[_row_spec(BN), _row_spec(BN), _row_spec(BN),
                  _row_spec(BN), _row_spec(BN),
                  _WSPEC, _WSPEC, _BSPEC, _WSPEC, _BSPEC, _WSPEC, _BSPEC,
                  _WSPEC, _WSPEC],
        out_specs=[_row_spec(BN)] * 3,
        out_shape=[jax.ShapeDtypeStruct((NP, D), _f32)] * 3,
        compiler_params=_CP,
    )(hn, pA[0], pA[1], pB[0], pB[1], V1n, V1a, c1, V2, c2, V3, c3,
      W1s, W1r)


def _dec(hn, D0, d0, D1, d1, D2, d2):
    def body(hn_r, D0_r, d0_r, D1_r, d1_r, D2_r, d2_r, out_r):
        h = jnp.maximum(_dot(hn_r[...], D0_r[...]) + d0_r[...], 0.0)
        h = jnp.maximum(_dot(h, D1_r[...]) + d1_r[...], 0.0)
        out_r[...] = _dot(h, D2_r[...]) + d2_r[...]

    return pl.pallas_call(
        body,
        grid=(NP // BN,),
        in_specs=[_row_spec(BN), _WSPEC, _BSPEC, _WSPEC, _BSPEC, _WSPEC,
                  _BSPEC],
        out_specs=[_row_spec(BN)],
        out_shape=[jax.ShapeDtypeStruct((NP, D), _f32)],
        compiler_params=_CP,
    )(hn, D0, d0, D1, d1, D2, d2)[0]


def _b(v):
    return v.reshape(1, D)


def kernel(x, edge_index, edge_attr, params):
    pad_e = EP - E
    epad = N + (jnp.arange(pad_e, dtype=jnp.int32) % (NP - N))
    send2 = jnp.concatenate([edge_index[0], epad]).reshape(2, NCHUNKS, CHUNK)
    recv2 = jnp.concatenate([edge_index[1], epad]).reshape(2, NCHUNKS, CHUNK)
    xp = jnp.pad(x, ((0, NP - N), (0, 0)))
    eap = jnp.pad(edge_attr, ((0, pad_e), (0, 4))).reshape(2, EH, 8)
    zeros = jnp.zeros((NP, D), _f32)

    (A0, a0), (A1, a1), (A2, a2) = params["enc_node"]
    (E0, e0), (E1, e1), (E2, e2) = params["enc_edge"]
    (D0, d0), (D1, d1), (D2, d2) = params["dec"]
    E0p = jnp.pad(E0, ((0, 4), (0, 0)))

    eW = []
    for s in range(STEPS):
        (W1, b1), (W2, b2), (W3, b3) = params["edge_mlp"][s]
        eW.append((W1[:D], W1[D:2 * D], W1[2 * D:], _b(b1), W2, _b(b2), W3,
                   _b(b3)))
    nW = []
    for s in range(STEPS):
        (V1, c1), (V2, c2), (V3, c3) = params["node_mlp"][s]
        nW.append((V1[:D], V1[D:], _b(c1), V2, _b(c2), V3, _b(c3)))

    hn, hs, hr = _enc_node(xp, A0, _b(a0), A1, _b(a1), A2, _b(a2),
                           eW[0][1], eW[0][2])
    heA = _enc_edge(eap[0], E0p, _b(e0), E1, _b(e1), E2, _b(e2))
    heB = _enc_edge(eap[1], E0p, _b(e0), E1, _b(e1), E2, _b(e2))

    for s in range(STEPS):
        W1e, _, _, b1, W2, b2, W3, b3 = eW[s]
        tSA, tRA = _sc_gather(hs, hr, send2[0], recv2[0])
        tSB, tRB = _sc_gather(hs, hr, send2[1], recv2[1])
        heA = _edge_step(heA, tSA, tRA, W1e, b1, W2, b2, W3, b3)
        pA = _sc_scatter(heA, recv2[0], zeros)
        heB = _edge_step(heB, tSB, tRB, W1e, b1, W2, b2, W3, b3)
        pB = _sc_scatter(heB, recv2[1], zeros)
        V1n, V1a, c1, V2, c2, V3, c3 = nW[s]
        nxt = eW[s + 1] if s + 1 < STEPS else eW[0]
        hn, hs, hr = _node_step(hn, pA, pB,
                                V1n, V1a, c1, V2, c2, V3, c3,
                                nxt[1], nxt[2])

    out = _dec(hn, D0, _b(d0), D1, _b(d1), D2, _b(d2))
    return out[:N]
